# initial kernel scaffold (unmeasured)
import jax
import jax.numpy as jnp
from jax import lax
from jax.experimental import pallas as pl
from jax.experimental.pallas import tpu as pltpu

N_DEV = 8
HPD = 8
DH = 128
SCALE = 0.08838834764831843
NEG = -1e9


def kernel(x, Wq, K_ext, V_ext, Wo):
    _, seq, d_model = x.shape
    kv = K_ext.shape[1]
    bf = jnp.bfloat16

    x2 = x[0].astype(bf)
    wq = Wq.astype(bf)
    wo = Wo.astype(bf)
    kt = K_ext[0].astype(bf).transpose(1, 2, 0)
    vt = V_ext[0].astype(bf).transpose(1, 0, 2)

    def body(x_ref, wq_ref, kt_ref, vt_ref, wo_ref, out_ref,
             comm_ref, qblk_ref, ctx_ref, mask_ref, send_sems, recv_sems):
        my = lax.axis_index("i")
        left = lax.rem(my - 1 + N_DEV, N_DEV)
        right = lax.rem(my + 1, N_DEV)

        barrier = pltpu.get_barrier_semaphore()
        for nbr in (left, right):
            pl.semaphore_signal(barrier, inc=1, device_id=(nbr,),
                                device_id_type=pl.DeviceIdType.MESH)
        pl.semaphore_wait(barrier, 2)

        rows = lax.broadcasted_iota(jnp.int32, (seq, kv), 0)
        cols = lax.broadcasted_iota(jnp.int32, (seq, kv), 1)
        qb = my * (seq // 64) + rows // 64
        kb = cols // 64
        keep = (qb == kb) | (kb == 0) | (lax.rem(qb + kb, 3) == 0)
        mask_ref[...] = jnp.where(keep, 0.0, NEG).astype(jnp.float32)

        out_ref[...] = jnp.zeros((seq, d_model), jnp.float32)

        comm_ref[N_DEV - 1, :d_model, :] = wq_ref[...]
        comm_ref[N_DEV - 1, d_model:, :] = wo_ref[...]

        def compute_block(origin, wq_blk, wo_blk):
            qf = jnp.dot(x_ref[...], wq_blk, preferred_element_type=jnp.float32)
            qblk_ref[...] = qf.astype(bf)
            base = origin * HPD

            def head_step(h, carry):
                g = base + h
                qh = pl.load(qblk_ref, (slice(None), pl.ds(h * DH, DH)))
                kh = pl.load(kt_ref, (pl.ds(g, 1), slice(None), slice(None)))
                s = jnp.dot(qh, kh.reshape(DH, kv),
                            preferred_element_type=jnp.float32)
                s = s * SCALE + mask_ref[...]
                m = jnp.max(s, axis=1, keepdims=True)
                w = jnp.exp(s - m)
                w = w / jnp.sum(w, axis=1, keepdims=True)
                vh = pl.load(vt_ref, (pl.ds(g, 1), slice(None), slice(None)))
                ch = jnp.dot(w.astype(bf), vh.reshape(kv, DH),
                             preferred_element_type=jnp.float32)
                pl.store(ctx_ref, (slice(None), pl.ds(h * DH, DH)),
                         ch.astype(bf))
                return carry

            lax.fori_loop(0, HPD, head_step, 0)
            out_ref[...] += jnp.dot(ctx_ref[...], wo_blk,
                                    preferred_element_type=jnp.float32)

        for h in range(N_DEV - 1):
            src_slot = N_DEV - 1 if h == 0 else h - 1
            rdma = pltpu.make_async_remote_copy(
                src_ref=comm_ref.at[src_slot],
                dst_ref=comm_ref.at[h],
                send_sem=send_sems.at[h],
                recv_sem=recv_sems.at[h],
                device_id=(right,),
                device_id_type=pl.DeviceIdType.MESH,
            )
            rdma.start()
            if h == 0:
                compute_block(my, wq_ref[...], wo_ref[...])
            else:
                origin = lax.rem(my - h + N_DEV, N_DEV)
                compute_block(origin, comm_ref[h - 1, :d_model, :],
                              comm_ref[h - 1, d_model:, :])
            rdma.wait()

        compute_block(right, comm_ref[N_DEV - 2, :d_model, :],
                      comm_ref[N_DEV - 2, d_model:, :])

    out = pl.pallas_call(
        body,
        out_shape=jax.ShapeDtypeStruct((seq, d_model), jnp.float32),
        in_specs=[pl.BlockSpec(memory_space=pltpu.VMEM)] * 5,
        out_specs=pl.BlockSpec(memory_space=pltpu.VMEM),
        scratch_shapes=[
            pltpu.VMEM((N_DEV, 2 * d_model, d_model), bf),
            pltpu.VMEM((seq, HPD * DH), bf),
            pltpu.VMEM((seq, HPD * DH), bf),
            pltpu.VMEM((seq, kv), jnp.float32),
            pltpu.SemaphoreType.DMA((N_DEV - 1,)),
            pltpu.SemaphoreType.DMA((N_DEV - 1,)),
        ],
        compiler_params=pltpu.CompilerParams(collective_id=0),
    )(x2, wq, kt, vt, wo)
    return out.reshape(1, seq, d_model)


# baseline (device time: 427398 ns/iter reference)
import jax
import jax.numpy as jnp
from jax import lax
from jax.experimental import pallas as pl
from jax.experimental.pallas import tpu as pltpu

N_DEV = 8
HPD = 8
DH = 128
SCALE = 0.08838834764831843
NEG = -1e9


def kernel(x, Wq, K_ext, V_ext, Wo):
    _, seq, d_model = x.shape
    kv = K_ext.shape[1]
    bf = jnp.bfloat16

    x2 = x[0].astype(bf)
    wq = Wq.astype(bf)
    wo = Wo.astype(bf)
    kt = K_ext[0].astype(bf).transpose(1, 2, 0)
    vt = V_ext[0].astype(bf).transpose(1, 0, 2)

    def body(x_ref, wq_ref, kt_ref, vt_ref, wo_ref, out_ref,
             comm_ref, qblk_ref, ctx_ref, mask_ref,
             kslab_ref, vslab_ref, send_sems, recv_sems, ksems, vsems):
        my = lax.axis_index("i")
        left = lax.rem(my - 1 + N_DEV, N_DEV)
        right = lax.rem(my + 1, N_DEV)

        barrier = pltpu.get_barrier_semaphore()
        for nbr in (left, right):
            pl.semaphore_signal(barrier, inc=1, device_id=(nbr,),
                                device_id_type=pl.DeviceIdType.MESH)
        pl.semaphore_wait(barrier, 2)

        def slab_copies(origin, slot):
            base = origin * HPD
            kc = pltpu.make_async_copy(
                kt_ref.at[pl.ds(base, HPD)],
                kslab_ref.at[pl.ds(slot * HPD, HPD)],
                ksems.at[slot])
            vc = pltpu.make_async_copy(
                vt_ref.at[pl.ds(base, HPD)],
                vslab_ref.at[pl.ds(slot * HPD, HPD)],
                vsems.at[slot])
            return kc, vc

        for c in slab_copies(my, 0):
            c.start()

        rows = lax.broadcasted_iota(jnp.int32, (seq, kv), 0)
        cols = lax.broadcasted_iota(jnp.int32, (seq, kv), 1)
        qb = my * (seq // 64) + rows // 64
        kb = cols // 64
        keep = (qb == kb) | (kb == 0) | (lax.rem(qb + kb, 3) == 0)
        mask_ref[...] = jnp.where(keep, 0.0, NEG).astype(bf)

        out_ref[...] = jnp.zeros((seq, d_model), jnp.float32)

        def compute_block(origin, slot, wq_blk, wo_blk):
            for c in slab_copies(origin, slot):
                c.wait()
            qf = jnp.dot(x_ref[...], wq_blk, preferred_element_type=jnp.float32)
            qblk_ref[...] = qf.astype(bf)

            def head_step(h, carry):
                idx = slot * HPD + h
                qh = qblk_ref[:, pl.ds(h * DH, DH)]
                kh = kslab_ref[pl.ds(idx, 1), :, :]
                s = jnp.dot(qh, kh.reshape(DH, kv),
                            preferred_element_type=jnp.float32)
                s = s * SCALE + mask_ref[...].astype(jnp.float32)
                m = jnp.max(s, axis=1, keepdims=True)
                w = jnp.exp(s - m)
                w = w / jnp.sum(w, axis=1, keepdims=True)
                vh = vslab_ref[pl.ds(idx, 1), :, :]
                ch = jnp.dot(w.astype(bf), vh.reshape(kv, DH),
                             preferred_element_type=jnp.float32)
                ctx_ref[:, pl.ds(h * DH, DH)] = ch.astype(bf)
                return carry

            lax.fori_loop(0, HPD, head_step, 0)
            out_ref[...] += jnp.dot(ctx_ref[...], wo_blk,
                                    preferred_element_type=jnp.float32)

        for h in range(N_DEV - 1):
            if h == 0:
                rdmas = [
                    pltpu.make_async_remote_copy(
                        src_ref=wq_ref,
                        dst_ref=comm_ref.at[0, pl.ds(0, d_model)],
                        send_sem=send_sems.at[0],
                        recv_sem=recv_sems.at[0],
                        device_id=(right,),
                        device_id_type=pl.DeviceIdType.MESH,
                    ),
                    pltpu.make_async_remote_copy(
                        src_ref=wo_ref,
                        dst_ref=comm_ref.at[0, pl.ds(d_model, d_model)],
                        send_sem=send_sems.at[N_DEV - 1],
                        recv_sem=recv_sems.at[N_DEV - 1],
                        device_id=(right,),
                        device_id_type=pl.DeviceIdType.MESH,
                    ),
                ]
            else:
                rdmas = [pltpu.make_async_remote_copy(
                    src_ref=comm_ref.at[h - 1],
                    dst_ref=comm_ref.at[h],
                    send_sem=send_sems.at[h],
                    recv_sem=recv_sems.at[h],
                    device_id=(right,),
                    device_id_type=pl.DeviceIdType.MESH,
                )]
            for r in rdmas:
                r.start()
            next_origin = lax.rem(my - h - 1 + N_DEV, N_DEV)
            for c in slab_copies(next_origin, (h + 1) % 2):
                c.start()
            if h == 0:
                compute_block(my, 0, wq_ref[...], wo_ref[...])
            else:
                origin = lax.rem(my - h + N_DEV, N_DEV)
                compute_block(origin, h % 2, comm_ref[h - 1, :d_model, :],
                              comm_ref[h - 1, d_model:, :])
            for r in rdmas:
                r.wait()

        compute_block(right, (N_DEV - 1) % 2, comm_ref[N_DEV - 2, :d_model, :],
                      comm_ref[N_DEV - 2, d_model:, :])

    out = pl.pallas_call(
        body,
        out_shape=jax.ShapeDtypeStruct((seq, d_model), jnp.float32),
        in_specs=[
            pl.BlockSpec(memory_space=pltpu.VMEM),
            pl.BlockSpec(memory_space=pltpu.VMEM),
            pl.BlockSpec(memory_space=pl.ANY),
            pl.BlockSpec(memory_space=pl.ANY),
            pl.BlockSpec(memory_space=pltpu.VMEM),
        ],
        out_specs=pl.BlockSpec(memory_space=pltpu.VMEM),
        scratch_shapes=[
            pltpu.VMEM((N_DEV - 1, 2 * d_model, d_model), bf),
            pltpu.VMEM((seq, HPD * DH), bf),
            pltpu.VMEM((seq, HPD * DH), bf),
            pltpu.VMEM((seq, kv), bf),
            pltpu.VMEM((2 * HPD, DH, kv), bf),
            pltpu.VMEM((2 * HPD, kv, DH), bf),
            pltpu.SemaphoreType.DMA((N_DEV,)),
            pltpu.SemaphoreType.DMA((N_DEV,)),
            pltpu.SemaphoreType.DMA((2,)),
            pltpu.SemaphoreType.DMA((2,)),
        ],
        compiler_params=pltpu.CompilerParams(
            collective_id=0, vmem_limit_bytes=65011712),
    )(x2, wq, kt, vt, wo)
    return out.reshape(1, seq, d_model)


# device time: 301947 ns/iter; 1.4155x vs baseline; 1.4155x over previous
import jax
import jax.numpy as jnp
from jax import lax
from jax.experimental import pallas as pl
from jax.experimental.pallas import tpu as pltpu

N_DEV = 8
HPD = 8
DH = 128
SCALE = 0.08838834764831843
NEG = -1e9


def kernel(x, Wq, K_ext, V_ext, Wo):
    _, seq, d_model = x.shape
    kv = K_ext.shape[1]
    bf = jnp.bfloat16

    x2 = x[0].astype(bf)
    wq = Wq.astype(bf)
    wo = Wo.astype(bf)
    kt = K_ext[0].astype(bf).transpose(1, 2, 0)
    vt = V_ext[0].astype(bf).transpose(1, 0, 2)

    def body(x_ref, wq_ref, kt_ref, vt_ref, wo_ref, out_ref,
             comm_ref, qblk_ref, ctx_ref, mask_ref,
             kslab_ref, vslab_ref, rs_send, rs_recv, ls_send, ls_recv,
             ksems, vsems):
        my = lax.axis_index("i")
        left = lax.rem(my - 1 + N_DEV, N_DEV)
        right = lax.rem(my + 1, N_DEV)

        barrier = pltpu.get_barrier_semaphore()
        for nbr in (left, right):
            pl.semaphore_signal(barrier, inc=1, device_id=(nbr,),
                                device_id_type=pl.DeviceIdType.MESH)
        pl.semaphore_wait(barrier, 2)

        def slab_copies(origin, slot):
            base = origin * HPD
            kc = pltpu.make_async_copy(
                kt_ref.at[pl.ds(base, HPD)],
                kslab_ref.at[pl.ds(slot * HPD, HPD)],
                ksems.at[slot])
            vc = pltpu.make_async_copy(
                vt_ref.at[pl.ds(base, HPD)],
                vslab_ref.at[pl.ds(slot * HPD, HPD)],
                vsems.at[slot])
            return kc, vc

        for c in slab_copies(my, 0):
            c.start()

        rows = lax.broadcasted_iota(jnp.int32, (seq, kv), 0)
        cols = lax.broadcasted_iota(jnp.int32, (seq, kv), 1)
        qb = my * (seq // 64) + rows // 64
        kb = cols // 64
        keep = (qb == kb) | (kb == 0) | (lax.rem(qb + kb, 3) == 0)
        mask_ref[...] = jnp.where(keep, 0.0, NEG).astype(bf)

        out_ref[...] = jnp.zeros((seq, d_model), jnp.float32)

        def compute_block(origin, slot, wq_blk, wo_blk):
            for c in slab_copies(origin, slot):
                c.wait()
            qf = jnp.dot(x_ref[...], wq_blk, preferred_element_type=jnp.float32)
            qblk_ref[...] = qf.astype(bf)

            def head_step(h, carry):
                idx = slot * HPD + h
                qh = qblk_ref[:, pl.ds(h * DH, DH)]
                kh = kslab_ref[pl.ds(idx, 1), :, :]
                s = jnp.dot(qh, kh.reshape(DH, kv),
                            preferred_element_type=jnp.float32)
                s = s * SCALE + mask_ref[...].astype(jnp.float32)
                m = jnp.max(s, axis=1, keepdims=True)
                w = jnp.exp(s - m)
                w = w / jnp.sum(w, axis=1, keepdims=True)
                vh = vslab_ref[pl.ds(idx, 1), :, :]
                ch = jnp.dot(w.astype(bf), vh.reshape(kv, DH),
                             preferred_element_type=jnp.float32)
                ctx_ref[:, pl.ds(h * DH, DH)] = ch.astype(bf)
                return carry

            lax.fori_loop(0, HPD, head_step, 0)
            out_ref[...] += jnp.dot(ctx_ref[...], wo_blk,
                                    preferred_element_type=jnp.float32)

        def own_send(dst_slot, sems_s, sems_r, base, dev):
            return [
                pltpu.make_async_remote_copy(
                    src_ref=wq_ref,
                    dst_ref=comm_ref.at[dst_slot, pl.ds(0, d_model)],
                    send_sem=sems_s.at[base],
                    recv_sem=sems_r.at[base],
                    device_id=(dev,),
                    device_id_type=pl.DeviceIdType.MESH,
                ),
                pltpu.make_async_remote_copy(
                    src_ref=wo_ref,
                    dst_ref=comm_ref.at[dst_slot, pl.ds(d_model, d_model)],
                    send_sem=sems_s.at[base + 1],
                    recv_sem=sems_r.at[base + 1],
                    device_id=(dev,),
                    device_id_type=pl.DeviceIdType.MESH,
                ),
            ]

        def fwd(src_slot, dst_slot, sems_s, sems_r, idx, dev):
            return [pltpu.make_async_remote_copy(
                src_ref=comm_ref.at[src_slot],
                dst_ref=comm_ref.at[dst_slot],
                send_sem=sems_s.at[idx],
                recv_sem=sems_r.at[idx],
                device_id=(dev,),
                device_id_type=pl.DeviceIdType.MESH,
            )]

        r0 = own_send(0, rs_send, rs_recv, 0, right)
        l0 = own_send(4, ls_send, ls_recv, 0, left)
        r1 = fwd(0, 1, rs_send, rs_recv, 2, right)
        r2 = fwd(1, 2, rs_send, rs_recv, 3, right)
        r3 = fwd(2, 3, rs_send, rs_recv, 4, right)
        l1 = fwd(4, 5, ls_send, ls_recv, 2, left)
        l2 = fwd(5, 6, ls_send, ls_recv, 3, left)

        def start(rs):
            for r in rs:
                r.start()

        def wait(rs):
            for r in rs:
                r.wait()

        def pos(k):
            return lax.rem(my + k + N_DEV, N_DEV)

        def comm_block(slot):
            return comm_ref[slot, :d_model, :], comm_ref[slot, d_model:, :]

        start(r0)
        start(l0)
        for c in slab_copies(pos(-1), 1):
            c.start()
        compute_block(my, 0, wq_ref[...], wo_ref[...])

        sched = [
            (-1, 0, r0, r1),
            (+1, 4, l0, l1),
            (-2, 1, r1, r2),
            (+2, 5, l1, l2),
            (-3, 2, r2, r3),
            (+3, 6, l2, None),
            (-4, 3, r3, None),
        ]
        for i, (off, slot, wait_r, start_r) in enumerate(sched):
            wait(wait_r)
            if start_r is not None:
                start(start_r)
            if i + 1 < len(sched):
                for c in slab_copies(pos(sched[i + 1][0]), i % 2):
                    c.start()
            wq_blk, wo_blk = comm_block(slot)
            compute_block(pos(off), (i + 1) % 2, wq_blk, wo_blk)

    out = pl.pallas_call(
        body,
        out_shape=jax.ShapeDtypeStruct((seq, d_model), jnp.float32),
        in_specs=[
            pl.BlockSpec(memory_space=pltpu.VMEM),
            pl.BlockSpec(memory_space=pltpu.VMEM),
            pl.BlockSpec(memory_space=pl.ANY),
            pl.BlockSpec(memory_space=pl.ANY),
            pl.BlockSpec(memory_space=pltpu.VMEM),
        ],
        out_specs=pl.BlockSpec(memory_space=pltpu.VMEM),
        scratch_shapes=[
            pltpu.VMEM((N_DEV - 1, 2 * d_model, d_model), bf),
            pltpu.VMEM((seq, HPD * DH), bf),
            pltpu.VMEM((seq, HPD * DH), bf),
            pltpu.VMEM((seq, kv), bf),
            pltpu.VMEM((2 * HPD, DH, kv), bf),
            pltpu.VMEM((2 * HPD, kv, DH), bf),
            pltpu.SemaphoreType.DMA((5,)),
            pltpu.SemaphoreType.DMA((5,)),
            pltpu.SemaphoreType.DMA((4,)),
            pltpu.SemaphoreType.DMA((4,)),
            pltpu.SemaphoreType.DMA((2,)),
            pltpu.SemaphoreType.DMA((2,)),
        ],
        compiler_params=pltpu.CompilerParams(
            collective_id=0, vmem_limit_bytes=65011712),
    )(x2, wq, kt, vt, wo)
    return out.reshape(1, seq, d_model)


# device time: 279508 ns/iter; 1.5291x vs baseline; 1.0803x over previous
import jax
import jax.numpy as jnp
from jax import lax
from jax.experimental import pallas as pl
from jax.experimental.pallas import tpu as pltpu

N_DEV = 8
HPD = 8
DH = 128
SCALE = 0.08838834764831843
NEG = -1e9


def kernel(x, Wq, K_ext, V_ext, Wo):
    _, seq, d_model = x.shape
    kv = K_ext.shape[1]
    bf = jnp.bfloat16

    x2 = x[0].astype(bf)
    wq = Wq.astype(bf)
    wo = Wo.astype(bf)
    kt = K_ext[0].astype(bf).transpose(1, 2, 0)
    vt = V_ext[0].astype(bf).transpose(1, 0, 2)

    def body(x_ref, wq_ref, kt_ref, vt_ref, wo_ref, out_ref,
             comm_ref, qblk_ref, ctx_ref, mask_ref,
             kslab_ref, vslab_ref, rs_send, rs_recv, ls_send, ls_recv,
             ksems, vsems):
        my = lax.axis_index("i")
        left = lax.rem(my - 1 + N_DEV, N_DEV)
        right = lax.rem(my + 1, N_DEV)

        barrier = pltpu.get_barrier_semaphore()
        for nbr in (left, right):
            pl.semaphore_signal(barrier, inc=1, device_id=(nbr,),
                                device_id_type=pl.DeviceIdType.MESH)
        pl.semaphore_wait(barrier, 2)

        def slab_copies(origin, slot):
            base = origin * HPD
            kc = pltpu.make_async_copy(
                kt_ref.at[pl.ds(base, HPD)],
                kslab_ref.at[pl.ds(slot * HPD, HPD)],
                ksems.at[slot])
            vc = pltpu.make_async_copy(
                vt_ref.at[pl.ds(base, HPD)],
                vslab_ref.at[pl.ds(slot * HPD, HPD)],
                vsems.at[slot])
            return kc, vc

        for c in slab_copies(my, 0):
            c.start()

        rows = lax.broadcasted_iota(jnp.int32, (seq, kv), 0)
        cols = lax.broadcasted_iota(jnp.int32, (seq, kv), 1)
        qb = my * (seq // 64) + rows // 64
        kb = cols // 64
        keep = (qb == kb) | (kb == 0) | (lax.rem(qb + kb, 3) == 0)
        mask_ref[...] = jnp.where(keep, 0.0, NEG).astype(bf)

        out_ref[...] = jnp.zeros((seq, d_model), jnp.float32)

        def compute_block(origin, slot, wq_blk, wo_blk):
            for c in slab_copies(origin, slot):
                c.wait()
            qf = jnp.dot(x_ref[...], wq_blk, preferred_element_type=jnp.float32)
            qblk_ref[...] = qf.astype(bf)

            def head_step(h, carry):
                idx = slot * HPD + h
                qh = qblk_ref[:, pl.ds(h * DH, DH)]
                kh = kslab_ref[pl.ds(idx, 1), :, :]
                s = jnp.dot(qh, kh.reshape(DH, kv),
                            preferred_element_type=jnp.float32)
                w = jnp.exp(s * SCALE + mask_ref[...].astype(jnp.float32))
                denom = jnp.sum(w, axis=1, keepdims=True)
                vh = vslab_ref[pl.ds(idx, 1), :, :]
                ch = jnp.dot(w.astype(bf), vh.reshape(kv, DH),
                             preferred_element_type=jnp.float32)
                ctx_ref[:, pl.ds(h * DH, DH)] = (ch / denom).astype(bf)
                return carry

            lax.fori_loop(0, HPD, head_step, 0)
            out_ref[...] += jnp.dot(ctx_ref[...], wo_blk,
                                    preferred_element_type=jnp.float32)

        def own_send(dst_slot, sems_s, sems_r, base, dev):
            return [
                pltpu.make_async_remote_copy(
                    src_ref=wq_ref,
                    dst_ref=comm_ref.at[dst_slot, pl.ds(0, d_model)],
                    send_sem=sems_s.at[base],
                    recv_sem=sems_r.at[base],
                    device_id=(dev,),
                    device_id_type=pl.DeviceIdType.MESH,
                ),
                pltpu.make_async_remote_copy(
                    src_ref=wo_ref,
                    dst_ref=comm_ref.at[dst_slot, pl.ds(d_model, d_model)],
                    send_sem=sems_s.at[base + 1],
                    recv_sem=sems_r.at[base + 1],
                    device_id=(dev,),
                    device_id_type=pl.DeviceIdType.MESH,
                ),
            ]

        def fwd(src_slot, dst_slot, sems_s, sems_r, idx, dev):
            return [pltpu.make_async_remote_copy(
                src_ref=comm_ref.at[src_slot],
                dst_ref=comm_ref.at[dst_slot],
                send_sem=sems_s.at[idx],
                recv_sem=sems_r.at[idx],
                device_id=(dev,),
                device_id_type=pl.DeviceIdType.MESH,
            )]

        r0 = own_send(0, rs_send, rs_recv, 0, right)
        l0 = own_send(4, ls_send, ls_recv, 0, left)
        r1 = fwd(0, 1, rs_send, rs_recv, 2, right)
        r2 = fwd(1, 2, rs_send, rs_recv, 3, right)
        r3 = fwd(2, 3, rs_send, rs_recv, 4, right)
        l1 = fwd(4, 5, ls_send, ls_recv, 2, left)
        l2 = fwd(5, 6, ls_send, ls_recv, 3, left)

        def start(rs):
            for r in rs:
                r.start()

        def wait(rs):
            for r in rs:
                r.wait()

        def pos(k):
            return lax.rem(my + k + N_DEV, N_DEV)

        def comm_block(slot):
            return comm_ref[slot, :d_model, :], comm_ref[slot, d_model:, :]

        start(r0)
        start(l0)
        for c in slab_copies(pos(-1), 1):
            c.start()
        compute_block(my, 0, wq_ref[...], wo_ref[...])

        sched = [
            (-1, 0, r0, r1),
            (+1, 4, l0, l1),
            (-2, 1, r1, r2),
            (+2, 5, l1, l2),
            (-3, 2, r2, r3),
            (+3, 6, l2, None),
            (-4, 3, r3, None),
        ]
        for i, (off, slot, wait_r, start_r) in enumerate(sched):
            wait(wait_r)
            if start_r is not None:
                start(start_r)
            if i + 1 < len(sched):
                for c in slab_copies(pos(sched[i + 1][0]), i % 2):
                    c.start()
            wq_blk, wo_blk = comm_block(slot)
            compute_block(pos(off), (i + 1) % 2, wq_blk, wo_blk)

    out = pl.pallas_call(
        body,
        out_shape=jax.ShapeDtypeStruct((seq, d_model), jnp.float32),
        in_specs=[
            pl.BlockSpec(memory_space=pltpu.VMEM),
            pl.BlockSpec(memory_space=pltpu.VMEM),
            pl.BlockSpec(memory_space=pl.ANY),
            pl.BlockSpec(memory_space=pl.ANY),
            pl.BlockSpec(memory_space=pltpu.VMEM),
        ],
        out_specs=pl.BlockSpec(memory_space=pltpu.VMEM),
        scratch_shapes=[
            pltpu.VMEM((N_DEV - 1, 2 * d_model, d_model), bf),
            pltpu.VMEM((seq, HPD * DH), bf),
            pltpu.VMEM((seq, HPD * DH), bf),
            pltpu.VMEM((seq, kv), bf),
            pltpu.VMEM((2 * HPD, DH, kv), bf),
            pltpu.VMEM((2 * HPD, kv, DH), bf),
            pltpu.SemaphoreType.DMA((5,)),
            pltpu.SemaphoreType.DMA((5,)),
            pltpu.SemaphoreType.DMA((4,)),
            pltpu.SemaphoreType.DMA((4,)),
            pltpu.SemaphoreType.DMA((2,)),
            pltpu.SemaphoreType.DMA((2,)),
        ],
        compiler_params=pltpu.CompilerParams(
            collective_id=0, vmem_limit_bytes=65011712),
    )(x2, wq, kt, vt, wo)
    return out.reshape(1, seq, d_model)


# device time: 279447 ns/iter; 1.5294x vs baseline; 1.0002x over previous
import jax
import jax.numpy as jnp
from jax import lax
from jax.experimental import pallas as pl
from jax.experimental.pallas import tpu as pltpu

N_DEV = 8
HPD = 8
DH = 128
SCALE = 0.08838834764831843
NEG = -1e9


def kernel(x, Wq, K_ext, V_ext, Wo):
    _, seq, d_model = x.shape
    kv = K_ext.shape[1]
    bf = jnp.bfloat16

    x2 = x[0].astype(bf)
    wq = Wq.astype(bf)
    wo = Wo.astype(bf)
    kt = K_ext[0].astype(bf).transpose(1, 2, 0)
    vt = V_ext[0].astype(bf).transpose(1, 0, 2)

    def body(x_ref, wq_ref, kt_ref, vt_ref, wo_ref, out_ref,
             comm_ref, qblk_ref, ctx_ref, mask_ref,
             kslab_ref, vslab_ref, rs_send, rs_recv, ls_send, ls_recv,
             ksems, vsems):
        my = lax.axis_index("i")
        left = lax.rem(my - 1 + N_DEV, N_DEV)
        right = lax.rem(my + 1, N_DEV)

        barrier = pltpu.get_barrier_semaphore()
        for nbr in (left, right):
            pl.semaphore_signal(barrier, inc=1, device_id=(nbr,),
                                device_id_type=pl.DeviceIdType.MESH)
        pl.semaphore_wait(barrier, 2)

        def slab_copies(origin, slot):
            base = origin * HPD
            kc = pltpu.make_async_copy(
                kt_ref.at[pl.ds(base, HPD)],
                kslab_ref.at[pl.ds(slot * HPD, HPD)],
                ksems.at[slot])
            vc = pltpu.make_async_copy(
                vt_ref.at[pl.ds(base, HPD)],
                vslab_ref.at[pl.ds(slot * HPD, HPD)],
                vsems.at[slot])
            return kc, vc

        for c in slab_copies(my, 0):
            c.start()

        rows = lax.broadcasted_iota(jnp.int32, (seq, kv), 0)
        cols = lax.broadcasted_iota(jnp.int32, (seq, kv), 1)
        qb = my * (seq // 64) + rows // 64
        kb = cols // 64
        keep = (qb == kb) | (kb == 0) | (lax.rem(qb + kb, 3) == 0)
        mask_ref[...] = jnp.where(keep, 0.0, NEG).astype(bf)

        out_ref[...] = jnp.zeros((seq, d_model), jnp.float32)

        def compute_block(origin, slot, wq_blk, wo_blk):
            for c in slab_copies(origin, slot):
                c.wait()
            qf = jnp.dot(x_ref[...], wq_blk, preferred_element_type=jnp.float32)
            qblk_ref[...] = qf.astype(bf)

            def head_step(h, carry):
                idx = slot * HPD + h
                qh = qblk_ref[:, pl.ds(h * DH, DH)]
                kh = kslab_ref[pl.ds(idx, 1), :, :]
                s = jnp.dot(qh, kh.reshape(DH, kv),
                            preferred_element_type=jnp.float32)
                w = jnp.exp((s * SCALE).astype(bf) + mask_ref[...])
                denom = jnp.sum(w, axis=1, keepdims=True, dtype=jnp.float32)
                vh = vslab_ref[pl.ds(idx, 1), :, :]
                ch = jnp.dot(w, vh.reshape(kv, DH),
                             preferred_element_type=jnp.float32)
                ctx_ref[:, pl.ds(h * DH, DH)] = (ch / denom).astype(bf)
                return carry

            lax.fori_loop(0, HPD, head_step, 0)
            out_ref[...] += jnp.dot(ctx_ref[...], wo_blk,
                                    preferred_element_type=jnp.float32)

        def own_send(dst_slot, sems_s, sems_r, base, dev):
            return [
                pltpu.make_async_remote_copy(
                    src_ref=wq_ref,
                    dst_ref=comm_ref.at[dst_slot, pl.ds(0, d_model)],
                    send_sem=sems_s.at[base],
                    recv_sem=sems_r.at[base],
                    device_id=(dev,),
                    device_id_type=pl.DeviceIdType.MESH,
                ),
                pltpu.make_async_remote_copy(
                    src_ref=wo_ref,
                    dst_ref=comm_ref.at[dst_slot, pl.ds(d_model, d_model)],
                    send_sem=sems_s.at[base + 1],
                    recv_sem=sems_r.at[base + 1],
                    device_id=(dev,),
                    device_id_type=pl.DeviceIdType.MESH,
                ),
            ]

        def fwd(src_slot, dst_slot, sems_s, sems_r, idx, dev):
            return [pltpu.make_async_remote_copy(
                src_ref=comm_ref.at[src_slot],
                dst_ref=comm_ref.at[dst_slot],
                send_sem=sems_s.at[idx],
                recv_sem=sems_r.at[idx],
                device_id=(dev,),
                device_id_type=pl.DeviceIdType.MESH,
            )]

        r0 = own_send(0, rs_send, rs_recv, 0, right)
        l0 = own_send(4, ls_send, ls_recv, 0, left)
        r1 = fwd(0, 1, rs_send, rs_recv, 2, right)
        r2 = fwd(1, 2, rs_send, rs_recv, 3, right)
        r3 = fwd(2, 3, rs_send, rs_recv, 4, right)
        l1 = fwd(4, 5, ls_send, ls_recv, 2, left)
        l2 = fwd(5, 6, ls_send, ls_recv, 3, left)

        def start(rs):
            for r in rs:
                r.start()

        def wait(rs):
            for r in rs:
                r.wait_recv()

        def pos(k):
            return lax.rem(my + k + N_DEV, N_DEV)

        def comm_block(slot):
            return comm_ref[slot, :d_model, :], comm_ref[slot, d_model:, :]

        start(r0)
        start(l0)
        for c in slab_copies(pos(-1), 1):
            c.start()
        compute_block(my, 0, wq_ref[...], wo_ref[...])

        sched = [
            (-1, 0, r0, r1),
            (+1, 4, l0, l1),
            (-2, 1, r1, r2),
            (+2, 5, l1, l2),
            (-3, 2, r2, r3),
            (+3, 6, l2, None),
            (-4, 3, r3, None),
        ]
        for i, (off, slot, wait_r, start_r) in enumerate(sched):
            wait(wait_r)
            if start_r is not None:
                start(start_r)
            if i + 1 < len(sched):
                for c in slab_copies(pos(sched[i + 1][0]), i % 2):
                    c.start()
            wq_blk, wo_blk = comm_block(slot)
            compute_block(pos(off), (i + 1) % 2, wq_blk, wo_blk)

        for rs in (r0, l0, r1, l1, r2, l2, r3):
            for r in rs:
                r.wait_send()

    out = pl.pallas_call(
        body,
        out_shape=jax.ShapeDtypeStruct((seq, d_model), jnp.float32),
        in_specs=[
            pl.BlockSpec(memory_space=pltpu.VMEM),
            pl.BlockSpec(memory_space=pltpu.VMEM),
            pl.BlockSpec(memory_space=pl.ANY),
            pl.BlockSpec(memory_space=pl.ANY),
            pl.BlockSpec(memory_space=pltpu.VMEM),
        ],
        out_specs=pl.BlockSpec(memory_space=pltpu.VMEM),
        scratch_shapes=[
            pltpu.VMEM((N_DEV - 1, 2 * d_model, d_model), bf),
            pltpu.VMEM((seq, HPD * DH), bf),
            pltpu.VMEM((seq, HPD * DH), bf),
            pltpu.VMEM((seq, kv), bf),
            pltpu.VMEM((2 * HPD, DH, kv), bf),
            pltpu.VMEM((2 * HPD, kv, DH), bf),
            pltpu.SemaphoreType.DMA((5,)),
            pltpu.SemaphoreType.DMA((5,)),
            pltpu.SemaphoreType.DMA((4,)),
            pltpu.SemaphoreType.DMA((4,)),
            pltpu.SemaphoreType.DMA((2,)),
            pltpu.SemaphoreType.DMA((2,)),
        ],
        compiler_params=pltpu.CompilerParams(
            collective_id=0, vmem_limit_bytes=65011712),
    )(x2, wq, kt, vt, wo)
    return out.reshape(1, seq, d_model)


# device time: 277501 ns/iter; 1.5402x vs baseline; 1.0070x over previous
import jax
import jax.numpy as jnp
from jax import lax
from jax.experimental import pallas as pl
from jax.experimental.pallas import tpu as pltpu

N_DEV = 8
HPD = 8
DH = 128
SCALE = 0.08838834764831843
NEG = -1e9


def kernel(x, Wq, K_ext, V_ext, Wo):
    _, seq, d_model = x.shape
    kv = K_ext.shape[1]
    bf = jnp.bfloat16

    x2 = x[0].astype(bf)
    wq = Wq.astype(bf)
    wo = Wo.astype(bf)
    kt = K_ext[0].astype(bf).transpose(1, 2, 0)
    vt = V_ext[0].astype(bf).transpose(1, 0, 2)

    def body(x_ref, wq_ref, kt_ref, vt_ref, wo_ref, out_ref,
             comm_ref, qblk_ref, ctx_ref, mask_ref,
             kslab_ref, vslab_ref, rs_send, rs_recv, ls_send, ls_recv,
             ksems, vsems):
        my = lax.axis_index("i")
        left = lax.rem(my - 1 + N_DEV, N_DEV)
        right = lax.rem(my + 1, N_DEV)

        barrier = pltpu.get_barrier_semaphore()
        for nbr in (left, right):
            pl.semaphore_signal(barrier, inc=1, device_id=(nbr,),
                                device_id_type=pl.DeviceIdType.MESH)
        pl.semaphore_wait(barrier, 2)

        def slab_copies(origin, slot):
            base = origin * HPD
            kc = pltpu.make_async_copy(
                kt_ref.at[pl.ds(base, HPD)],
                kslab_ref.at[pl.ds(slot * HPD, HPD)],
                ksems.at[slot])
            vc = pltpu.make_async_copy(
                vt_ref.at[pl.ds(base, HPD)],
                vslab_ref.at[pl.ds(slot * HPD, HPD)],
                vsems.at[slot])
            return kc, vc

        for c in slab_copies(my, 0):
            c.start()

        rows = lax.broadcasted_iota(jnp.int32, (seq, kv), 0)
        cols = lax.broadcasted_iota(jnp.int32, (seq, kv), 1)
        qb = my * (seq // 64) + rows // 64
        kb = cols // 64
        keep = (qb == kb) | (kb == 0) | (lax.rem(qb + kb, 3) == 0)
        mask_ref[...] = jnp.where(keep, 0.0, NEG).astype(bf)

        out_ref[...] = jnp.zeros((seq, d_model), jnp.float32)

        def compute_block(origin, slot, wq_blk, wo_blk):
            for c in slab_copies(origin, slot):
                c.wait()
            qblk_ref[...] = jnp.dot(x_ref[...], wq_blk,
                                    preferred_element_type=jnp.float32
                                    ).astype(bf)

            def head_step(h, carry):
                idx = slot * HPD + h
                qh = qblk_ref[:, pl.ds(h * DH, DH)]
                kh = kslab_ref[pl.ds(idx, 1), :, :]
                s = jnp.dot(qh, kh.reshape(DH, kv),
                            preferred_element_type=jnp.float32)
                w = jnp.exp((s * SCALE).astype(bf) + mask_ref[...])
                denom = jnp.sum(w, axis=1, keepdims=True, dtype=jnp.float32)
                vh = vslab_ref[pl.ds(idx, 1), :, :]
                ch = jnp.dot(w, vh.reshape(kv, DH),
                             preferred_element_type=jnp.float32)
                ctx_ref[:, pl.ds(h * DH, DH)] = (ch / denom).astype(bf)
                return carry

            lax.fori_loop(0, HPD, head_step, 0, unroll=4)
            out_ref[...] += jnp.dot(ctx_ref[...], wo_blk,
                                    preferred_element_type=jnp.float32)

        def own_send(dst_slot, sems_s, sems_r, base, dev):
            return [
                pltpu.make_async_remote_copy(
                    src_ref=wq_ref,
                    dst_ref=comm_ref.at[dst_slot, pl.ds(0, d_model)],
                    send_sem=sems_s.at[base],
                    recv_sem=sems_r.at[base],
                    device_id=(dev,),
                    device_id_type=pl.DeviceIdType.MESH,
                ),
                pltpu.make_async_remote_copy(
                    src_ref=wo_ref,
                    dst_ref=comm_ref.at[dst_slot, pl.ds(d_model, d_model)],
                    send_sem=sems_s.at[base + 1],
                    recv_sem=sems_r.at[base + 1],
                    device_id=(dev,),
                    device_id_type=pl.DeviceIdType.MESH,
                ),
            ]

        def fwd(src_slot, dst_slot, sems_s, sems_r, idx, dev):
            return [pltpu.make_async_remote_copy(
                src_ref=comm_ref.at[src_slot],
                dst_ref=comm_ref.at[dst_slot],
                send_sem=sems_s.at[idx],
                recv_sem=sems_r.at[idx],
                device_id=(dev,),
                device_id_type=pl.DeviceIdType.MESH,
            )]

        r0 = own_send(0, rs_send, rs_recv, 0, right)
        l0 = own_send(4, ls_send, ls_recv, 0, left)
        r1 = fwd(0, 1, rs_send, rs_recv, 2, right)
        r2 = fwd(1, 2, rs_send, rs_recv, 3, right)
        r3 = fwd(2, 3, rs_send, rs_recv, 4, right)
        l1 = fwd(4, 5, ls_send, ls_recv, 2, left)
        l2 = fwd(5, 6, ls_send, ls_recv, 3, left)

        def start(rs):
            for r in rs:
                r.start()

        def wait(rs):
            for r in rs:
                r.wait_recv()

        def pos(k):
            return lax.rem(my + k + N_DEV, N_DEV)

        def comm_block(slot):
            return comm_ref[slot, :d_model, :], comm_ref[slot, d_model:, :]

        start(r0)
        start(l0)
        for c in slab_copies(pos(-1), 1):
            c.start()
        compute_block(my, 0, wq_ref[...], wo_ref[...])

        sched = [
            (-1, 0, r0, r1),
            (+1, 4, l0, l1),
            (-2, 1, r1, r2),
            (+2, 5, l1, l2),
            (-3, 2, r2, r3),
            (+3, 6, l2, None),
            (-4, 3, r3, None),
        ]
        for i, (off, slot, wait_r, start_r) in enumerate(sched):
            wait(wait_r)
            if start_r is not None:
                start(start_r)
            if i + 1 < len(sched):
                for c in slab_copies(pos(sched[i + 1][0]), i % 2):
                    c.start()
            wq_blk, wo_blk = comm_block(slot)
            compute_block(pos(off), (i + 1) % 2, wq_blk, wo_blk)

        for rs in (r0, l0, r1, l1, r2, l2, r3):
            for r in rs:
                r.wait_send()

    out = pl.pallas_call(
        body,
        out_shape=jax.ShapeDtypeStruct((seq, d_model), jnp.float32),
        in_specs=[
            pl.BlockSpec(memory_space=pltpu.VMEM),
            pl.BlockSpec(memory_space=pltpu.VMEM),
            pl.BlockSpec(memory_space=pl.ANY),
            pl.BlockSpec(memory_space=pl.ANY),
            pl.BlockSpec(memory_space=pltpu.VMEM),
        ],
        out_specs=pl.BlockSpec(memory_space=pltpu.VMEM),
        scratch_shapes=[
            pltpu.VMEM((N_DEV - 1, 2 * d_model, d_model), bf),
            pltpu.VMEM((seq, HPD * DH), bf),
            pltpu.VMEM((seq, HPD * DH), bf),
            pltpu.VMEM((seq, kv), bf),
            pltpu.VMEM((2 * HPD, DH, kv), bf),
            pltpu.VMEM((2 * HPD, kv, DH), bf),
            pltpu.SemaphoreType.DMA((5,)),
            pltpu.SemaphoreType.DMA((5,)),
            pltpu.SemaphoreType.DMA((4,)),
            pltpu.SemaphoreType.DMA((4,)),
            pltpu.SemaphoreType.DMA((2,)),
            pltpu.SemaphoreType.DMA((2,)),
        ],
        compiler_params=pltpu.CompilerParams(
            collective_id=0, vmem_limit_bytes=65011712),
    )(x2, wq, kt, vt, wo)
    return out.reshape(1, seq, d_model)
